# Initial kernel scaffold; baseline (speedup 1.0000x reference)
import jax
import jax.numpy as jnp
from jax.experimental import pallas as pl


def kernel(x_s, x_t, edge_index, edge_attr, u, batch_s, batch_t, W1, b1, W2, b2):
    raise NotImplementedError("write your pallas kernel here")


# speedup vs baseline: 4.8172x; 4.8172x over previous
import functools

import jax
import jax.numpy as jnp
from jax import lax
from jax.experimental import pallas as pl
from jax.experimental.pallas import tpu as pltpu
from jax.experimental.pallas import tpu_sc as plsc

N = 1600000
B = 4096
F_S = 10
F_T = 5
W = 16

NC = 2
NS = 16
NW = NC * NS

GROUP = 128
GPT = 392
CHUNK_G = 4
CHUNK = GROUP * CHUNK_G
NCHUNK = GPT // CHUNK_G
NGROUPS = N // GROUP
SLICE = B // NS


def _sc_segment_sums(xs_flat, xt_flat, ids_s2d, ids_t2d, z16):
    mesh = plsc.VectorSubcoreMesh(core_axis_name="c", subcore_axis_name="s")

    @functools.partial(
        pl.kernel,
        out_type=[
            jax.ShapeDtypeStruct((NC, B, W), jnp.float32),
            jax.ShapeDtypeStruct((NC, B, W), jnp.float32),
        ],
        mesh=mesh,
        compiler_params=pltpu.CompilerParams(use_tc_tiling_on_sc=False),
        scratch_types=[
            pltpu.VMEM((CHUNK * F_S + W,), jnp.float32),
            pltpu.VMEM((CHUNK * F_T + W,), jnp.float32),
            pltpu.VMEM((CHUNK, W), jnp.float32),
            pltpu.VMEM((CHUNK, W), jnp.float32),
            pltpu.VMEM((CHUNK_G, GROUP), jnp.int32),
            pltpu.VMEM((CHUNK_G, GROUP), jnp.int32),
            pltpu.VMEM_SHARED((B, W), jnp.float32),
            pltpu.VMEM_SHARED((B, W), jnp.float32),
        ],
    )
    def seg_kernel(xs_hbm, xt_hbm, ids_s_hbm, ids_t_hbm, z16_hbm,
                   ps_hbm, pt_hbm,
                   xs_buf, xt_buf, wide_s, wide_t, ibs, ibt,
                   acc_s, acc_t):
        core = lax.axis_index("c")
        sid = lax.axis_index("s")
        wid = sid * NC + core
        rz = sid * SLICE

        pltpu.sync_copy(z16_hbm.at[pl.ds(rz, SLICE)], acc_s.at[pl.ds(rz, SLICE)])
        pltpu.sync_copy(z16_hbm.at[pl.ds(rz, SLICE)], acc_t.at[pl.ds(rz, SLICE)])
        plsc.subcore_barrier()

        lane = lax.iota(jnp.int32, 16)
        is_cnt_s = lane == F_S
        is_cnt_t = lane == F_T
        one = jnp.full((16,), 1.0, jnp.float32)

        def chunk_body(c, carry):
            g0 = wid * GPT + c * CHUNK_G

            @pl.when(g0 + CHUNK_G <= NGROUPS)
            def _():
                row0 = g0 * GROUP
                pltpu.sync_copy(xs_hbm.at[pl.ds(row0 * F_S, CHUNK * F_S)],
                                xs_buf.at[pl.ds(0, CHUNK * F_S)])
                pltpu.sync_copy(xt_hbm.at[pl.ds(row0 * F_T, CHUNK * F_T)],
                                xt_buf.at[pl.ds(0, CHUNK * F_T)])
                pltpu.sync_copy(ids_s_hbm.at[pl.ds(g0, CHUNK_G)], ibs)
                pltpu.sync_copy(ids_t_hbm.at[pl.ds(g0, CHUNK_G)], ibt)

                def widen(r, carry2):
                    vs = xs_buf[pl.ds(r * F_S, 16)]
                    wide_s[r, :] = jnp.where(is_cnt_s, one, vs)
                    vt = xt_buf[pl.ds(r * F_T, 16)]
                    wide_t[r, :] = jnp.where(is_cnt_t, one, vt)
                    return carry2

                lax.fori_loop(0, CHUNK, widen, 0)

                for j in range(CHUNK_G):
                    pltpu.sync_copy(wide_s.at[pl.ds(j * GROUP, GROUP)],
                                    acc_s.at[ibs.at[j]], add=True)
                    pltpu.sync_copy(wide_t.at[pl.ds(j * GROUP, GROUP)],
                                    acc_t.at[ibt.at[j]], add=True)

            return carry

        lax.fori_loop(0, NCHUNK, chunk_body, 0)
        plsc.subcore_barrier()

        pltpu.sync_copy(acc_s.at[pl.ds(rz, SLICE)], ps_hbm.at[core, pl.ds(rz, SLICE)])
        pltpu.sync_copy(acc_t.at[pl.ds(rz, SLICE)], pt_hbm.at[core, pl.ds(rz, SLICE)])

    return seg_kernel(xs_flat, xt_flat, ids_s2d, ids_t2d, z16)


def _mlp_body(ps, pt, u, w1a, w1b, w1c, b1, w2, b2, out):
    acc_s = ps[0] + ps[1]
    acc_t = pt[0] + pt[1]
    mean_s = acc_s[:, :F_S] / jnp.maximum(acc_s[:, F_S:F_S + 1], 1.0)
    mean_t = acc_t[:, :F_T] / jnp.maximum(acc_t[:, F_T:F_T + 1], 1.0)
    h = (jnp.dot(u[...], w1a[...], preferred_element_type=jnp.float32)
         + jnp.dot(mean_s, w1b[...], preferred_element_type=jnp.float32)
         + jnp.dot(mean_t, w1c[...], preferred_element_type=jnp.float32)
         + b1[...])
    h = jnp.where(h >= 0, h, 0.1 * h)
    out[...] = jnp.dot(h, w2[...], preferred_element_type=jnp.float32) + b2[...]


def kernel(x_s, x_t, edge_index, edge_attr, u, batch_s, batch_t, W1, b1, W2, b2):
    del edge_index, edge_attr

    xs_flat = x_s.reshape(N * F_S)
    xt_flat = x_t.reshape(N * F_T)
    ids_s2d = batch_s.astype(jnp.int32).reshape(NGROUPS, GROUP)
    ids_t2d = batch_t.astype(jnp.int32).reshape(NGROUPS, GROUP)
    z16 = jnp.zeros((B, W), jnp.float32)

    ps, pt = _sc_segment_sums(xs_flat, xt_flat, ids_s2d, ids_t2d, z16)

    out = pl.pallas_call(
        _mlp_body,
        out_shape=jax.ShapeDtypeStruct((B, F_S), jnp.float32),
    )(ps, pt, u,
      W1[:F_S], W1[F_S:F_S + F_S], W1[F_S + F_S:], b1.reshape(1, F_S),
      W2, b2.reshape(1, F_S))
    return out
